# Initial kernel scaffold; baseline (speedup 1.0000x reference)
import jax
import jax.numpy as jnp
from jax.experimental import pallas as pl


def kernel(x, Wr, W1, W2):
    raise NotImplementedError("write your pallas kernel here")


# speedup vs baseline: 2.6039x; 2.6039x over previous
import functools

import jax
import jax.numpy as jnp
from jax import lax
from jax.experimental import pallas as pl
from jax.experimental.pallas import tpu as pltpu
from jax.experimental.pallas import tpu_sc as plsc

C_DIM = 768
E_NUM = 8
K_TOP = 2
H_DIM = 3072
N_TOK = 2048
NK = N_TOK * K_TOP

BLK = 128
P_PAD = 5120
NBLK = P_PAD // BLK

NC, NS, L = 2, 16, 16
NW = NC * NS
ROWS_PT = P_PAD // NW
GCH = 2
GROWS = ROWS_PT // GCH
TOK_PT = N_TOK // NW
LCH = C_DIM // L

_SQRT_HALF = 0.7071067811865476


def _routing_body(x_ref, wr_ref, w0_ref, w1_ref, i0_ref, i1_ref):
    logits = jnp.dot(x_ref[...], wr_ref[...],
                     preferred_element_type=jnp.float32)
    iota = lax.broadcasted_iota(jnp.int32, logits.shape, 1)
    m1 = jnp.max(logits, axis=1, keepdims=True)
    i0 = jnp.min(jnp.where(logits == m1, iota, E_NUM), axis=1)
    masked = jnp.where(iota == i0[:, None], -jnp.inf, logits)
    m2 = jnp.max(masked, axis=1, keepdims=True)
    i1 = jnp.min(jnp.where(masked == m2, iota, E_NUM), axis=1)
    r = jnp.exp((m2 - m1)[:, 0])
    w0 = 1.0 / (1.0 + r)
    w0_ref[...] = w0
    w1_ref[...] = 1.0 - w0
    i0_ref[...] = i0
    i1_ref[...] = i1


_routing_call = pl.pallas_call(
    _routing_body,
    out_shape=[
        jax.ShapeDtypeStruct((N_TOK,), jnp.float32),
        jax.ShapeDtypeStruct((N_TOK,), jnp.float32),
        jax.ShapeDtypeStruct((N_TOK,), jnp.int32),
        jax.ShapeDtypeStruct((N_TOK,), jnp.int32),
    ],
)


def _sc_gather(x_hbm, tok_hbm, out_hbm, idx_v, rows_v, sem):
    wid = lax.axis_index("s") * NC + lax.axis_index("c")
    pltpu.sync_copy(tok_hbm.at[wid], idx_v)
    for j in range(GCH):
        pltpu.async_copy(x_hbm.at[idx_v.at[j]], rows_v, sem).wait()
        pltpu.sync_copy(
            rows_v, out_hbm.at[pl.ds(wid * ROWS_PT + j * GROWS, GROWS)])


@functools.cache
def _gather_call():
    return pl.kernel(
        _sc_gather,
        mesh=plsc.VectorSubcoreMesh(core_axis_name="c", subcore_axis_name="s"),
        out_type=jax.ShapeDtypeStruct((P_PAD, C_DIM), jnp.float32),
        scratch_types=[
            pltpu.VMEM((GCH, GROWS), jnp.int32),
            pltpu.VMEM((GROWS, C_DIM), jnp.float32),
            pltpu.SemaphoreType.DMA,
        ],
    )


def _ffn_body(be_ref, x_ref, w1_ref, w2_ref, wp_ref, o_ref):
    h = jnp.dot(x_ref[...], w1_ref[0], preferred_element_type=jnp.float32)
    h = 0.5 * h * (1.0 + lax.erf(h * _SQRT_HALF))
    y = jnp.dot(h, w2_ref[0], preferred_element_type=jnp.float32)
    o_ref[...] = y * wp_ref[...]


_ffn_call = pl.pallas_call(
    _ffn_body,
    grid_spec=pltpu.PrefetchScalarGridSpec(
        num_scalar_prefetch=1,
        grid=(NBLK,),
        in_specs=[
            pl.BlockSpec((BLK, C_DIM), lambda i, be: (i, 0)),
            pl.BlockSpec((1, C_DIM, H_DIM), lambda i, be: (be[i], 0, 0)),
            pl.BlockSpec((1, H_DIM, C_DIM), lambda i, be: (be[i], 0, 0)),
            pl.BlockSpec((BLK, 1), lambda i, be: (i, 0)),
        ],
        out_specs=pl.BlockSpec((BLK, C_DIM), lambda i, be: (i, 0)),
    ),
    out_shape=jax.ShapeDtypeStruct((P_PAD, C_DIM), jnp.float32),
    compiler_params=pltpu.CompilerParams(
        dimension_semantics=("arbitrary",),
        vmem_limit_bytes=100 * 1024 * 1024,
    ),
)


def _sc_combine(yg_hbm, p0_hbm, p1_hbm, out_hbm,
                i0_v, i1_v, b0_v, b1_v, sem0, sem1):
    wid = lax.axis_index("s") * NC + lax.axis_index("c")
    base = wid * TOK_PT
    pltpu.sync_copy(p0_hbm.at[wid], i0_v)
    pltpu.sync_copy(p1_hbm.at[wid], i1_v)
    c0 = pltpu.async_copy(yg_hbm.at[i0_v], b0_v, sem0)
    c1 = pltpu.async_copy(yg_hbm.at[i1_v], b1_v, sem1)
    c0.wait()
    c1.wait()

    def _row(r, carry):
        for j in range(LCH):
            sl = pl.ds(j * L, L)
            b0_v[r, sl] = b0_v[r, sl] + b1_v[r, sl]
        return carry

    lax.fori_loop(0, TOK_PT, _row, 0)
    pltpu.sync_copy(b0_v, out_hbm.at[pl.ds(base, TOK_PT)])


@functools.cache
def _combine_call():
    return pl.kernel(
        _sc_combine,
        mesh=plsc.VectorSubcoreMesh(core_axis_name="c", subcore_axis_name="s"),
        out_type=jax.ShapeDtypeStruct((N_TOK, C_DIM), jnp.float32),
        scratch_types=[
            pltpu.VMEM((TOK_PT,), jnp.int32),
            pltpu.VMEM((TOK_PT,), jnp.int32),
            pltpu.VMEM((TOK_PT, C_DIM), jnp.float32),
            pltpu.VMEM((TOK_PT, C_DIM), jnp.float32),
            pltpu.SemaphoreType.DMA,
            pltpu.SemaphoreType.DMA,
        ],
    )


def kernel(x, Wr, W1, W2):
    Bb, Tt, C = x.shape
    x_flat = x.reshape(-1, C)

    w0, w1, i0, i1 = _routing_call(x_flat, Wr)

    e_flat = jnp.stack([i0, i1], axis=1).reshape(-1)
    w_flat = jnp.stack([w0, w1], axis=1).reshape(-1)
    order = jnp.argsort(e_flat)
    e_sorted = e_flat[order]
    counts = jnp.zeros((E_NUM,), jnp.int32).at[e_flat].add(1)
    starts = jnp.cumsum(counts) - counts
    pc = ((counts + BLK - 1) // BLK) * BLK
    pstarts = jnp.cumsum(pc) - pc
    rank = jnp.arange(NK, dtype=jnp.int32) - starts[e_sorted]
    pos = (pstarts[e_sorted] + rank).astype(jnp.int32)
    tok_pad = jnp.zeros((P_PAD,), jnp.int32).at[pos].set(
        (order // K_TOP).astype(jnp.int32))
    w_pad = jnp.zeros((P_PAD,), jnp.float32).at[pos].set(w_flat[order])
    inv = jnp.zeros((NK,), jnp.int32).at[order].set(pos)
    p0 = inv[0::2]
    p1 = inv[1::2]
    be = (jnp.searchsorted(
        pstarts, jnp.arange(NBLK, dtype=jnp.int32) * BLK, side='right')
        - 1).astype(jnp.int32)

    xg = _gather_call()(x_flat, tok_pad.reshape(NW, GCH, GROWS))
    yg = _ffn_call(be, xg, W1, W2, w_pad[:, None])
    out = _combine_call()(yg, p0.reshape(NW, TOK_PT), p1.reshape(NW, TOK_PT))
    return out.reshape(Bb, Tt, C)


# speedup vs baseline: 2.8561x; 1.0969x over previous
import functools

import jax
import jax.numpy as jnp
from jax import lax
from jax.experimental import pallas as pl
from jax.experimental.pallas import tpu as pltpu
from jax.experimental.pallas import tpu_sc as plsc

C_DIM = 768
E_NUM = 8
K_TOP = 2
H_DIM = 3072
N_TOK = 2048
NK = N_TOK * K_TOP

BLK = 128
P_PAD = 5120
NBLK = P_PAD // BLK

NC, NS, L = 2, 16, 16
NW = NC * NS
ROWS_PT = P_PAD // NW
GCH = 4
GROWS = ROWS_PT // GCH
GNB = 3
TOK_PT = N_TOK // NW
LCH = C_DIM // L

_SQRT_HALF = 0.7071067811865476


def _routing_body(x_ref, wr_ref, w0_ref, w1_ref, i0_ref, i1_ref):
    logits = jnp.dot(x_ref[...], wr_ref[...],
                     preferred_element_type=jnp.float32)
    iota = lax.broadcasted_iota(jnp.int32, logits.shape, 1)
    m1 = jnp.max(logits, axis=1, keepdims=True)
    i0 = jnp.min(jnp.where(logits == m1, iota, E_NUM), axis=1)
    masked = jnp.where(iota == i0[:, None], -jnp.inf, logits)
    m2 = jnp.max(masked, axis=1, keepdims=True)
    i1 = jnp.min(jnp.where(masked == m2, iota, E_NUM), axis=1)
    r = jnp.exp((m2 - m1)[:, 0])
    w0 = 1.0 / (1.0 + r)
    w0_ref[...] = w0
    w1_ref[...] = 1.0 - w0
    i0_ref[...] = i0
    i1_ref[...] = i1


_routing_call = pl.pallas_call(
    _routing_body,
    out_shape=[
        jax.ShapeDtypeStruct((N_TOK,), jnp.float32),
        jax.ShapeDtypeStruct((N_TOK,), jnp.float32),
        jax.ShapeDtypeStruct((N_TOK,), jnp.int32),
        jax.ShapeDtypeStruct((N_TOK,), jnp.int32),
    ],
)


def _sc_gather(x_hbm, tok_hbm, out_hbm, idx_v,
               b0, b1, b2, g0, g1, g2, s0, s1, s2):
    wid = lax.axis_index("s") * NC + lax.axis_index("c")
    pltpu.sync_copy(tok_hbm.at[wid], idx_v)
    bufs = (b0, b1, b2)
    gsem = (g0, g1, g2)
    wsem = (s0, s1, s2)
    base = wid * ROWS_PT
    for j in range(min(GNB, GCH)):
        pltpu.async_copy(x_hbm.at[idx_v.at[j]], bufs[j], gsem[j])
    wb = [None] * GCH
    for j in range(GCH):
        k = j % GNB
        pltpu.make_async_copy(x_hbm.at[idx_v.at[j]], bufs[k], gsem[k]).wait()
        wb[j] = pltpu.async_copy(
            bufs[k], out_hbm.at[pl.ds(base + j * GROWS, GROWS)], wsem[k])
        if j + GNB < GCH:
            wb[j].wait()
            pltpu.async_copy(x_hbm.at[idx_v.at[j + GNB]], bufs[k], gsem[k])
            wb[j] = None
    for j in range(GCH):
        if wb[j] is not None:
            wb[j].wait()


@functools.cache
def _gather_call():
    return pl.kernel(
        _sc_gather,
        mesh=plsc.VectorSubcoreMesh(core_axis_name="c", subcore_axis_name="s"),
        out_type=jax.ShapeDtypeStruct((P_PAD, C_DIM), jnp.float32),
        scratch_types=[
            pltpu.VMEM((GCH, GROWS), jnp.int32),
            pltpu.VMEM((GROWS, C_DIM), jnp.float32),
            pltpu.VMEM((GROWS, C_DIM), jnp.float32),
            pltpu.VMEM((GROWS, C_DIM), jnp.float32),
            pltpu.SemaphoreType.DMA,
            pltpu.SemaphoreType.DMA,
            pltpu.SemaphoreType.DMA,
            pltpu.SemaphoreType.DMA,
            pltpu.SemaphoreType.DMA,
            pltpu.SemaphoreType.DMA,
        ],
    )


_DOT_DIMS = (((1,), (0,)), ((), ()))


def _ffn_body(be_ref, x_ref, w1_ref, w2_ref, wp_ref, o_ref):
    h = lax.dot_general(x_ref[...], w1_ref[0], _DOT_DIMS,
                        preferred_element_type=jnp.float32)
    h = 0.5 * h * (1.0 + lax.erf(h * _SQRT_HALF))
    y = lax.dot_general(h, w2_ref[0], _DOT_DIMS,
                        preferred_element_type=jnp.float32)
    o_ref[...] = y * wp_ref[...]


_ffn_call = pl.pallas_call(
    _ffn_body,
    grid_spec=pltpu.PrefetchScalarGridSpec(
        num_scalar_prefetch=1,
        grid=(NBLK,),
        in_specs=[
            pl.BlockSpec((BLK, C_DIM), lambda i, be: (i, 0)),
            pl.BlockSpec((1, C_DIM, H_DIM), lambda i, be: (be[i], 0, 0)),
            pl.BlockSpec((1, H_DIM, C_DIM), lambda i, be: (be[i], 0, 0)),
            pl.BlockSpec((BLK, 1), lambda i, be: (i, 0)),
        ],
        out_specs=pl.BlockSpec((BLK, C_DIM), lambda i, be: (i, 0)),
    ),
    out_shape=jax.ShapeDtypeStruct((P_PAD, C_DIM), jnp.float32),
    compiler_params=pltpu.CompilerParams(
        dimension_semantics=("arbitrary",),
        vmem_limit_bytes=100 * 1024 * 1024,
    ),
)


def _sc_combine(yg_hbm, p0_hbm, p1_hbm, out_hbm,
                i0_v, i1_v, b0_v, b1_v, sem0, sem1):
    wid = lax.axis_index("s") * NC + lax.axis_index("c")
    base = wid * TOK_PT
    pltpu.sync_copy(p0_hbm.at[wid], i0_v)
    pltpu.sync_copy(p1_hbm.at[wid], i1_v)
    c0 = pltpu.async_copy(yg_hbm.at[i0_v], b0_v, sem0)
    c1 = pltpu.async_copy(yg_hbm.at[i1_v], b1_v, sem1)
    c0.wait()
    c1.wait()

    def _row(r, carry):
        for j in range(LCH):
            sl = pl.ds(j * L, L)
            b0_v[r, sl] = b0_v[r, sl] + b1_v[r, sl]
        return carry

    lax.fori_loop(0, TOK_PT, _row, 0)
    pltpu.sync_copy(b0_v, out_hbm.at[pl.ds(base, TOK_PT)])


@functools.cache
def _combine_call():
    return pl.kernel(
        _sc_combine,
        mesh=plsc.VectorSubcoreMesh(core_axis_name="c", subcore_axis_name="s"),
        out_type=jax.ShapeDtypeStruct((N_TOK, C_DIM), jnp.float32),
        scratch_types=[
            pltpu.VMEM((TOK_PT,), jnp.int32),
            pltpu.VMEM((TOK_PT,), jnp.int32),
            pltpu.VMEM((TOK_PT, C_DIM), jnp.float32),
            pltpu.VMEM((TOK_PT, C_DIM), jnp.float32),
            pltpu.SemaphoreType.DMA,
            pltpu.SemaphoreType.DMA,
        ],
    )


def kernel(x, Wr, W1, W2):
    Bb, Tt, C = x.shape
    x_flat = x.reshape(-1, C)

    w0, w1, i0, i1 = _routing_call(x_flat, Wr)

    e_flat = jnp.stack([i0, i1], axis=1).reshape(-1)
    w_flat = jnp.stack([w0, w1], axis=1).reshape(-1)
    onehot = (e_flat[:, None] ==
              jnp.arange(E_NUM, dtype=e_flat.dtype)[None, :]).astype(jnp.int32)
    csum = jnp.cumsum(onehot, axis=0)
    counts = csum[-1]
    rank = jnp.sum(onehot * csum, axis=1) - 1
    pc = ((counts + BLK - 1) // BLK) * BLK
    pstarts = jnp.cumsum(pc) - pc
    pos = (jnp.take(pstarts, e_flat) + rank).astype(jnp.int32)
    tok_pad = jnp.zeros((P_PAD,), jnp.int32).at[pos].set(
        jnp.arange(NK, dtype=jnp.int32) // K_TOP)
    w_pad = jnp.zeros((P_PAD,), jnp.float32).at[pos].set(w_flat)
    p0 = pos[0::2]
    p1 = pos[1::2]
    blk_starts = jnp.arange(NBLK, dtype=jnp.int32) * BLK
    be = (jnp.sum(
        blk_starts[:, None] >= pstarts[None, :], axis=1) - 1).astype(jnp.int32)

    xg = _gather_call()(x_flat, tok_pad.reshape(NW, GCH, GROWS))
    yg = _ffn_call(be, xg, W1, W2, w_pad[:, None])
    out = _combine_call()(yg, p0.reshape(NW, TOK_PT), p1.reshape(NW, TOK_PT))
    return out.reshape(Bb, Tt, C)


# speedup vs baseline: 3.8007x; 1.3307x over previous
import functools

import jax
import jax.numpy as jnp
from jax import lax
from jax.experimental import pallas as pl
from jax.experimental.pallas import tpu as pltpu
from jax.experimental.pallas import tpu_sc as plsc

C_DIM = 768
E_NUM = 8
K_TOP = 2
H_DIM = 3072
N_TOK = 2048
NK = N_TOK * K_TOP

BLK = 128
P_PAD = 5120
NBLK = P_PAD // BLK

NC, NS, L = 2, 16, 16
NW = NC * NS
ROWS_PT = P_PAD // NW
GCH = 10
GROWS = ROWS_PT // GCH
GNB = 2
TOK_PT = N_TOK // NW
LCH = C_DIM // L

_SQRT_HALF = 0.7071067811865476


def _routing_body(x_ref, wr_ref, w0_ref, w1_ref, i0_ref, i1_ref):
    logits = jnp.dot(x_ref[...], wr_ref[...],
                     preferred_element_type=jnp.float32)
    iota = lax.broadcasted_iota(jnp.int32, logits.shape, 1)
    m1 = jnp.max(logits, axis=1, keepdims=True)
    i0 = jnp.min(jnp.where(logits == m1, iota, E_NUM), axis=1)
    masked = jnp.where(iota == i0[:, None], -jnp.inf, logits)
    m2 = jnp.max(masked, axis=1, keepdims=True)
    i1 = jnp.min(jnp.where(masked == m2, iota, E_NUM), axis=1)
    r = jnp.exp((m2 - m1)[:, 0])
    w0 = 1.0 / (1.0 + r)
    w0_ref[...] = w0
    w1_ref[...] = 1.0 - w0
    i0_ref[...] = i0
    i1_ref[...] = i1


_routing_call = pl.pallas_call(
    _routing_body,
    out_shape=[
        jax.ShapeDtypeStruct((N_TOK,), jnp.float32),
        jax.ShapeDtypeStruct((N_TOK,), jnp.float32),
        jax.ShapeDtypeStruct((N_TOK,), jnp.int32),
        jax.ShapeDtypeStruct((N_TOK,), jnp.int32),
    ],
)


def _sc_dispatch(x_hbm, p0_hbm, p1_hbm, out_hbm,
                 i0_v, i1_v, rows_v, sg, s0, s1):
    wid = lax.axis_index("s") * NC + lax.axis_index("c")
    base = wid * TOK_PT
    pltpu.sync_copy(p0_hbm.at[wid], i0_v)
    pltpu.sync_copy(p1_hbm.at[wid], i1_v)
    pltpu.async_copy(x_hbm.at[pl.ds(base, TOK_PT)], rows_v, sg).wait()
    c0 = pltpu.async_copy(rows_v, out_hbm.at[i0_v], s0)
    c1 = pltpu.async_copy(rows_v, out_hbm.at[i1_v], s1)
    c0.wait()
    c1.wait()


@functools.cache
def _gather_call():
    return pl.kernel(
        _sc_dispatch,
        mesh=plsc.VectorSubcoreMesh(core_axis_name="c", subcore_axis_name="s"),
        out_type=jax.ShapeDtypeStruct((P_PAD, C_DIM), jnp.float32),
        scratch_types=[
            pltpu.VMEM((TOK_PT,), jnp.int32),
            pltpu.VMEM((TOK_PT,), jnp.int32),
            pltpu.VMEM((TOK_PT, C_DIM), jnp.float32),
            pltpu.SemaphoreType.DMA,
            pltpu.SemaphoreType.DMA,
            pltpu.SemaphoreType.DMA,
        ],
    )


_DOT_DIMS = (((1,), (0,)), ((), ()))


def _ffn_body(be_ref, x_ref, w1_ref, w2_ref, wp_ref, o_ref):
    h = lax.dot_general(x_ref[...], w1_ref[0], _DOT_DIMS,
                        preferred_element_type=jnp.float32)
    h = 0.5 * h * (1.0 + lax.erf(h * _SQRT_HALF))
    y = lax.dot_general(h, w2_ref[0], _DOT_DIMS,
                        preferred_element_type=jnp.float32)
    o_ref[...] = y * wp_ref[...]


_ffn_call = pl.pallas_call(
    _ffn_body,
    grid_spec=pltpu.PrefetchScalarGridSpec(
        num_scalar_prefetch=1,
        grid=(NBLK,),
        in_specs=[
            pl.BlockSpec((BLK, C_DIM), lambda i, be: (i, 0)),
            pl.BlockSpec((1, C_DIM, H_DIM), lambda i, be: (be[i], 0, 0)),
            pl.BlockSpec((1, H_DIM, C_DIM), lambda i, be: (be[i], 0, 0)),
            pl.BlockSpec((BLK, 1), lambda i, be: (i, 0)),
        ],
        out_specs=pl.BlockSpec((BLK, C_DIM), lambda i, be: (i, 0)),
    ),
    out_shape=jax.ShapeDtypeStruct((P_PAD, C_DIM), jnp.float32),
    compiler_params=pltpu.CompilerParams(
        dimension_semantics=("arbitrary",),
        vmem_limit_bytes=100 * 1024 * 1024,
    ),
)


def _sc_combine(yg_hbm, p0_hbm, p1_hbm, out_hbm,
                i0_v, i1_v, b0_v, b1_v, sem0, sem1):
    wid = lax.axis_index("s") * NC + lax.axis_index("c")
    base = wid * TOK_PT
    pltpu.sync_copy(p0_hbm.at[wid], i0_v)
    pltpu.sync_copy(p1_hbm.at[wid], i1_v)
    c0 = pltpu.async_copy(yg_hbm.at[i0_v], b0_v, sem0)
    c1 = pltpu.async_copy(yg_hbm.at[i1_v], b1_v, sem1)
    c0.wait()
    c1.wait()

    def _row(r, carry):
        for j in range(LCH):
            sl = pl.ds(j * L, L)
            b0_v[r, sl] = b0_v[r, sl] + b1_v[r, sl]
        return carry

    lax.fori_loop(0, TOK_PT, _row, 0)
    pltpu.sync_copy(b0_v, out_hbm.at[pl.ds(base, TOK_PT)])


@functools.cache
def _combine_call():
    return pl.kernel(
        _sc_combine,
        mesh=plsc.VectorSubcoreMesh(core_axis_name="c", subcore_axis_name="s"),
        out_type=jax.ShapeDtypeStruct((N_TOK, C_DIM), jnp.float32),
        scratch_types=[
            pltpu.VMEM((TOK_PT,), jnp.int32),
            pltpu.VMEM((TOK_PT,), jnp.int32),
            pltpu.VMEM((TOK_PT, C_DIM), jnp.float32),
            pltpu.VMEM((TOK_PT, C_DIM), jnp.float32),
            pltpu.SemaphoreType.DMA,
            pltpu.SemaphoreType.DMA,
        ],
    )


def kernel(x, Wr, W1, W2):
    Bb, Tt, C = x.shape
    x_flat = x.reshape(-1, C)

    w0, w1, i0, i1 = _routing_call(x_flat, Wr)

    e_flat = jnp.stack([i0, i1], axis=1).reshape(-1)
    w_flat = jnp.stack([w0, w1], axis=1).reshape(-1)
    onehot = (e_flat[:, None] ==
              jnp.arange(E_NUM, dtype=e_flat.dtype)[None, :]).astype(jnp.int32)
    csum = jnp.cumsum(onehot, axis=0)
    counts = csum[-1]
    rank = jnp.sum(onehot * csum, axis=1) - 1
    pc = ((counts + BLK - 1) // BLK) * BLK
    pstarts = jnp.cumsum(pc) - pc
    pos = (jnp.take(pstarts, e_flat) + rank).astype(jnp.int32)
    w_pad = jnp.zeros((P_PAD,), jnp.float32).at[pos].set(w_flat)
    p0 = pos[0::2]
    p1 = pos[1::2]
    blk_starts = jnp.arange(NBLK, dtype=jnp.int32) * BLK
    be = (jnp.sum(
        blk_starts[:, None] >= pstarts[None, :], axis=1) - 1).astype(jnp.int32)

    p0m = p0.reshape(NW, TOK_PT)
    p1m = p1.reshape(NW, TOK_PT)
    xg = _gather_call()(x_flat, p0m, p1m)
    yg = _ffn_call(be, xg, W1, W2, w_pad[:, None])
    out = _combine_call()(yg, p0m, p1m)
    return out.reshape(Bb, Tt, C)
